# Initial kernel scaffold; baseline (speedup 1.0000x reference)
import jax
import jax.numpy as jnp
from jax.experimental import pallas as pl


def kernel(hidden_states, position_ids, ln1_w, Wq, Wk, Wv, Wo, ln2_w, W_router, expert_bias, Wg, Wu, Wd, Wsg, Wsu, Wsd):
    raise NotImplementedError("write your pallas kernel here")


# speedup vs baseline: 1.4618x; 1.4618x over previous
import functools
import math

import jax
import jax.numpy as jnp
import numpy as np
from jax.experimental import pallas as pl
from jax.experimental.pallas import tpu as pltpu

B, S, D = 1, 2048, 2048
H, HKV, DH = 16, 4, 128
E, K, I = 8, 2, 1024
EPS, THETA = 1e-05, 10000.0

BT = 256
NTB = S // BT

F32 = jnp.float32
BF16 = jnp.bfloat16


def _rms(xf, w):
    var = jnp.mean(xf * xf, axis=-1, keepdims=True)
    return w * (xf * jax.lax.rsqrt(var + EPS))


def _rope(x, cos_t, sin_t, width):
    n = width // DH
    cos = jnp.concatenate([cos_t] * n, axis=1)
    sin = jnp.concatenate([sin_t] * n, axis=1)
    lane = jax.lax.broadcasted_iota(jnp.int32, (BT, width), 1)
    left = jnp.roll(x, -64, axis=1)
    right = jnp.roll(x, 64, axis=1)
    rot = jnp.where((lane % 128) < 64, -left, right)
    return x * cos + rot * sin


def _k1(x_ref, cos_ref, sin_ref, ln1_ref, wq_ref, wk_ref, wv_ref,
        q_ref, k_ref, v_ref):
    h = _rms(x_ref[...].astype(F32), ln1_ref[...].astype(F32))
    cos_t = cos_ref[...]
    sin_t = sin_ref[...]
    q = jax.lax.dot(h, wq_ref[...], preferred_element_type=F32)
    k = jax.lax.dot(h, wk_ref[...], preferred_element_type=F32)
    v = jax.lax.dot(h, wv_ref[...], preferred_element_type=F32)
    q_ref[...] = _rope(q, cos_t, sin_t, H * DH).astype(BF16)
    k_ref[...] = _rope(k, cos_t, sin_t, HKV * DH).astype(BF16)
    v_ref[...] = v.astype(BF16)


BQ = 1024
BK = 1024
SCALE = np.float32(0.0883883461)


def _k2(q_ref, k_ref, v_ref, o_ref):
    qb = pl.program_id(1)
    q = q_ref[...]
    qpos = qb * BQ + jax.lax.broadcasted_iota(jnp.int32, (BQ, BK), 0)
    m_prev = jnp.full((BQ, 1), -jnp.inf, F32)
    l_prev = jnp.zeros((BQ, 1), F32)
    acc = jnp.zeros((BQ, DH), F32)
    for kb in range(S // BK):
        k = k_ref[kb * BK:(kb + 1) * BK, :]
        v = v_ref[kb * BK:(kb + 1) * BK, :]
        s = jax.lax.dot_general(q, k, (((1,), (1,)), ((), ())),
                                preferred_element_type=F32)
        s = s * SCALE
        kpos = kb * BK + jax.lax.broadcasted_iota(jnp.int32, (BQ, BK), 1)
        s = jnp.where(kpos <= qpos, s, -1e9)
        m_cur = jnp.max(s, axis=-1, keepdims=True)
        m_new = jnp.maximum(m_prev, m_cur)
        delta = jnp.where(m_prev == m_new, 0.0, m_prev - m_new)
        p = jnp.exp(s - m_new)
        l_cur = jnp.sum(p, axis=-1, keepdims=True)
        corr_l = jnp.exp(delta) * l_prev
        l_new = corr_l + l_cur
        o = jax.lax.dot(p.astype(BF16), v, preferred_element_type=F32)
        o = o + acc * corr_l
        acc = o * (1.0 / l_new)
        m_prev, l_prev = m_new, l_new
    o_ref[...] = acc.astype(BF16)


def _k3(ctx_ref, x_ref, wo_ref, ln2_ref, wr_ref, bias_ref,
        xa_ref, h2_ref, cmb_ref):
    attn = jax.lax.dot(ctx_ref[...], wo_ref[...], preferred_element_type=F32)
    xa = x_ref[...].astype(F32) + attn
    xa_ref[...] = xa
    h2 = _rms(xa, ln2_ref[...].astype(F32))
    h2_ref[...] = h2.astype(BF16)
    logits = jax.lax.dot(h2.astype(BF16), wr_ref[...].astype(BF16),
                         preferred_element_type=F32)
    sc = jax.nn.sigmoid(logits)
    biased = sc + bias_ref[...].astype(F32)
    lane = jax.lax.broadcasted_iota(jnp.int32, biased.shape, 1)

    def top1(b):
        m = jnp.max(b, axis=-1, keepdims=True)
        eq = b == m
        first = jnp.min(jnp.where(eq, lane, E), axis=-1, keepdims=True)
        return lane == first

    is0 = top1(biased)
    w0 = jnp.sum(jnp.where(is0, sc, 0.0), axis=-1, keepdims=True)
    is1 = top1(jnp.where(is0, -jnp.inf, biased))
    w1 = jnp.sum(jnp.where(is1, sc, 0.0), axis=-1, keepdims=True)
    wsum = w0 + w1 + 1e-9
    cmb_ref[...] = (jnp.where(is0, w0, 0.0) + jnp.where(is1, w1, 0.0)) / wsum


def _k5(h2_ref, cmb_ref, wg_ref, wu_ref, wd_ref, out_ref, acc_ref):
    e = pl.program_id(0)
    tb = pl.program_id(1)
    h2 = h2_ref[...]
    gate = jax.lax.dot(h2, wg_ref[0], preferred_element_type=F32)
    up = jax.lax.dot(h2, wu_ref[0], preferred_element_type=F32)
    ymid = (gate * jax.nn.sigmoid(gate) * up).astype(BF16)
    y = jax.lax.dot(ymid, wd_ref[0], preferred_element_type=F32)
    cmb = cmb_ref[...]
    lane = jax.lax.broadcasted_iota(jnp.int32, cmb.shape, 1)
    w = jnp.sum(jnp.where(lane == e, cmb, 0.0), axis=-1, keepdims=True)
    y = y.astype(BF16).astype(F32) * w.astype(BF16).astype(F32)
    sl = pl.ds(tb * BT, BT)

    @pl.when(e == 0)
    def _():
        acc_ref[sl, :] = y

    @pl.when(e > 0)
    def _():
        acc_ref[sl, :] = acc_ref[sl, :] + y

    out_ref[...] = acc_ref[sl, :]


def _k6(xa_ref, routed_ref, h2_ref, wsg_ref, wsu_ref, wsd_ref, out_ref):
    h2 = h2_ref[...]
    gate = jax.lax.dot(h2, wsg_ref[...], preferred_element_type=F32)
    up = jax.lax.dot(h2, wsu_ref[...], preferred_element_type=F32)
    ymid = (gate * jax.nn.sigmoid(gate) * up).astype(BF16)
    y = jax.lax.dot(ymid, wsd_ref[...], preferred_element_type=F32)
    out_ref[...] = xa_ref[...] + routed_ref[...] + y


@jax.jit
def kernel(hidden_states, position_ids, ln1_w, Wq, Wk, Wv, Wo, ln2_w,
           W_router, expert_bias, Wg, Wu, Wd, Wsg, Wsu, Wsd):
    x = hidden_states.reshape(S, D)
    inv_freq = 1.0 / (THETA ** (jnp.arange(0, DH, 2, dtype=F32) / DH))
    ang = position_ids.reshape(S).astype(F32)[:, None] * inv_freq
    cos_t = jnp.concatenate([jnp.cos(ang), jnp.cos(ang)], axis=-1)
    sin_t = jnp.concatenate([jnp.sin(ang), jnp.sin(ang)], axis=-1)
    ln1 = ln1_w.reshape(1, D)
    ln2 = ln2_w.reshape(1, D)
    bias = expert_bias.reshape(1, E)
    wg, wu, wd = (w.astype(BF16) for w in (Wg, Wu, Wd))
    wsg, wsu, wsd = (w.astype(BF16) for w in (Wsg, Wsu, Wsd))

    q, k, v = pl.pallas_call(
        _k1,
        grid=(NTB,),
        in_specs=[
            pl.BlockSpec((BT, D), lambda t: (t, 0)),
            pl.BlockSpec((BT, DH), lambda t: (t, 0)),
            pl.BlockSpec((BT, DH), lambda t: (t, 0)),
            pl.BlockSpec((1, D), lambda t: (0, 0)),
            pl.BlockSpec((D, H * DH), lambda t: (0, 0)),
            pl.BlockSpec((D, HKV * DH), lambda t: (0, 0)),
            pl.BlockSpec((D, HKV * DH), lambda t: (0, 0)),
        ],
        out_specs=[
            pl.BlockSpec((BT, H * DH), lambda t: (t, 0)),
            pl.BlockSpec((BT, HKV * DH), lambda t: (t, 0)),
            pl.BlockSpec((BT, HKV * DH), lambda t: (t, 0)),
        ],
        out_shape=[
            jax.ShapeDtypeStruct((S, H * DH), BF16),
            jax.ShapeDtypeStruct((S, HKV * DH), BF16),
            jax.ShapeDtypeStruct((S, HKV * DH), BF16),
        ],
        compiler_params=pltpu.CompilerParams(
            vmem_limit_bytes=100 * 1024 * 1024),
    )(x, cos_t, sin_t, ln1, Wq, Wk, Wv)

    ctx = pl.pallas_call(
        _k2,
        grid=(H, S // BQ),
        in_specs=[
            pl.BlockSpec((BQ, DH), lambda h, t: (t, h)),
            pl.BlockSpec((S, DH), lambda h, t: (0, h // (H // HKV))),
            pl.BlockSpec((S, DH), lambda h, t: (0, h // (H // HKV))),
        ],
        out_specs=pl.BlockSpec((BQ, DH), lambda h, t: (t, h)),
        out_shape=jax.ShapeDtypeStruct((S, H * DH), BF16),
        compiler_params=pltpu.CompilerParams(
            vmem_limit_bytes=100 * 1024 * 1024),
    )(q, k, v)

    xa, h2, cmb = pl.pallas_call(
        _k3,
        grid=(NTB,),
        in_specs=[
            pl.BlockSpec((BT, H * DH), lambda t: (t, 0)),
            pl.BlockSpec((BT, D), lambda t: (t, 0)),
            pl.BlockSpec((H * DH, D), lambda t: (0, 0)),
            pl.BlockSpec((1, D), lambda t: (0, 0)),
            pl.BlockSpec((D, E), lambda t: (0, 0)),
            pl.BlockSpec((1, E), lambda t: (0, 0)),
        ],
        out_specs=[
            pl.BlockSpec((BT, D), lambda t: (t, 0)),
            pl.BlockSpec((BT, D), lambda t: (t, 0)),
            pl.BlockSpec((BT, E), lambda t: (t, 0)),
        ],
        out_shape=[
            jax.ShapeDtypeStruct((S, D), F32),
            jax.ShapeDtypeStruct((S, D), BF16),
            jax.ShapeDtypeStruct((S, E), F32),
        ],
        compiler_params=pltpu.CompilerParams(
            vmem_limit_bytes=100 * 1024 * 1024),
    )(ctx, x, Wo, ln2, W_router, bias)

    routed = pl.pallas_call(
        _k5,
        grid=(E, NTB),
        in_specs=[
            pl.BlockSpec((BT, D), lambda e, t: (t, 0)),
            pl.BlockSpec((BT, E), lambda e, t: (t, 0)),
            pl.BlockSpec((1, D, I), lambda e, t: (e, 0, 0)),
            pl.BlockSpec((1, D, I), lambda e, t: (e, 0, 0)),
            pl.BlockSpec((1, I, D), lambda e, t: (e, 0, 0)),
        ],
        out_specs=pl.BlockSpec((BT, D), lambda e, t: (t, 0)),
        out_shape=jax.ShapeDtypeStruct((S, D), F32),
        scratch_shapes=[pltpu.VMEM((S, D), F32)],
        compiler_params=pltpu.CompilerParams(
            dimension_semantics=("arbitrary", "arbitrary"),
            vmem_limit_bytes=100 * 1024 * 1024),
    )(h2, cmb, wg, wu, wd)

    out = pl.pallas_call(
        _k6,
        grid=(NTB,),
        in_specs=[
            pl.BlockSpec((BT, D), lambda t: (t, 0)),
            pl.BlockSpec((BT, D), lambda t: (t, 0)),
            pl.BlockSpec((BT, D), lambda t: (t, 0)),
            pl.BlockSpec((D, I), lambda t: (0, 0)),
            pl.BlockSpec((D, I), lambda t: (0, 0)),
            pl.BlockSpec((I, D), lambda t: (0, 0)),
        ],
        out_specs=pl.BlockSpec((BT, D), lambda t: (t, 0)),
        out_shape=jax.ShapeDtypeStruct((S, D), F32),
        compiler_params=pltpu.CompilerParams(
            vmem_limit_bytes=100 * 1024 * 1024),
    )(xa, routed, h2, wsg, wsu, wsd)

    return out.reshape(B, S, D)


# speedup vs baseline: 1.4988x; 1.0253x over previous
import functools
import math

import jax
import jax.numpy as jnp
import numpy as np
from jax.experimental import pallas as pl
from jax.experimental.pallas import tpu as pltpu

B, S, D = 1, 2048, 2048
H, HKV, DH = 16, 4, 128
E, K, I = 8, 2, 1024
EPS, THETA = 1e-05, 10000.0

BT = 256
NTB = S // BT

F32 = jnp.float32
BF16 = jnp.bfloat16


def _rms(xf, w):
    var = jnp.mean(xf * xf, axis=-1, keepdims=True)
    return w * (xf * jax.lax.rsqrt(var + EPS))


def _rope(x, cos_t, sin_t, width):
    n = width // DH
    cos = jnp.concatenate([cos_t] * n, axis=1)
    sin = jnp.concatenate([sin_t] * n, axis=1)
    lane = jax.lax.broadcasted_iota(jnp.int32, (BT, width), 1)
    left = jnp.roll(x, -64, axis=1)
    right = jnp.roll(x, 64, axis=1)
    rot = jnp.where((lane % 128) < 64, -left, right)
    return x * cos + rot * sin


def _k1(x_ref, cos_ref, sin_ref, ln1_ref, wq_ref, wk_ref, wv_ref,
        q_ref, k_ref, v_ref):
    h = _rms(x_ref[...].astype(F32), ln1_ref[...].astype(F32))
    cos_t = cos_ref[...]
    sin_t = sin_ref[...]
    q = jax.lax.dot(h, wq_ref[...], preferred_element_type=F32)
    k = jax.lax.dot(h, wk_ref[...], preferred_element_type=F32)
    v = jax.lax.dot(h, wv_ref[...], preferred_element_type=F32)
    q_ref[...] = _rope(q, cos_t, sin_t, H * DH).astype(BF16)
    k_ref[...] = _rope(k, cos_t, sin_t, HKV * DH).astype(BF16)
    v_ref[...] = v.astype(BF16)


BQ = 1024
BK = 1024
SCALE = np.float32(0.0883883461)


def _k2(q_ref, k_ref, v_ref, o_ref):
    qb = pl.program_id(1)
    q = q_ref[...]
    qpos = qb * BQ + jax.lax.broadcasted_iota(jnp.int32, (BQ, BK), 0)
    m_prev = jnp.full((BQ, 1), -jnp.inf, F32)
    l_prev = jnp.zeros((BQ, 1), F32)
    acc = jnp.zeros((BQ, DH), F32)
    for kb in range(S // BK):
        k = k_ref[kb * BK:(kb + 1) * BK, :]
        v = v_ref[kb * BK:(kb + 1) * BK, :]
        s = jax.lax.dot_general(q, k, (((1,), (1,)), ((), ())),
                                preferred_element_type=F32)
        s = s * SCALE
        kpos = kb * BK + jax.lax.broadcasted_iota(jnp.int32, (BQ, BK), 1)
        s = jnp.where(kpos <= qpos, s, -1e9)
        m_cur = jnp.max(s, axis=-1, keepdims=True)
        m_new = jnp.maximum(m_prev, m_cur)
        delta = jnp.where(m_prev == m_new, 0.0, m_prev - m_new)
        p = jnp.exp(s - m_new)
        l_cur = jnp.sum(p, axis=-1, keepdims=True)
        corr_l = jnp.exp(delta) * l_prev
        l_new = corr_l + l_cur
        o = jax.lax.dot(p.astype(BF16), v, preferred_element_type=F32)
        o = o + acc * corr_l
        acc = o * (1.0 / l_new)
        m_prev, l_prev = m_new, l_new
    o_ref[...] = acc.astype(BF16)


def _k3(ctx_ref, x_ref, wo_ref, ln2_ref, wr_ref, bias_ref,
        xa_ref, h2_ref, cmb_ref):
    attn = jax.lax.dot(ctx_ref[...], wo_ref[...], preferred_element_type=F32)
    xa = x_ref[...].astype(F32) + attn
    xa_ref[...] = xa
    h2 = _rms(xa, ln2_ref[...].astype(F32))
    h2_ref[...] = h2.astype(BF16)
    logits = jax.lax.dot(h2.astype(BF16), wr_ref[...].astype(BF16),
                         preferred_element_type=F32)
    sc = jax.nn.sigmoid(logits)
    biased = sc + bias_ref[...].astype(F32)
    lane = jax.lax.broadcasted_iota(jnp.int32, biased.shape, 1)

    def top1(b):
        m = jnp.max(b, axis=-1, keepdims=True)
        eq = b == m
        first = jnp.min(jnp.where(eq, lane, E), axis=-1, keepdims=True)
        return lane == first

    is0 = top1(biased)
    w0 = jnp.sum(jnp.where(is0, sc, 0.0), axis=-1, keepdims=True)
    is1 = top1(jnp.where(is0, -jnp.inf, biased))
    w1 = jnp.sum(jnp.where(is1, sc, 0.0), axis=-1, keepdims=True)
    wsum = w0 + w1 + 1e-9
    cmb_ref[...] = (jnp.where(is0, w0, 0.0) + jnp.where(is1, w1, 0.0)) / wsum


def _k5(h2_ref, cmb_ref, wg_ref, wu_ref, wd_ref, out_ref):
    e = pl.program_id(0)
    tb = pl.program_id(1)
    sl = pl.ds(tb * BT, BT)
    h2 = h2_ref[sl, :]
    gate = jax.lax.dot(h2, wg_ref[0], preferred_element_type=F32)
    up = jax.lax.dot(h2, wu_ref[0], preferred_element_type=F32)
    ymid = (gate * jax.nn.sigmoid(gate) * up).astype(BF16)
    y = jax.lax.dot(ymid, wd_ref[0], preferred_element_type=F32)
    cmb = cmb_ref[sl, :]
    lane = jax.lax.broadcasted_iota(jnp.int32, cmb.shape, 1)
    w = jnp.sum(jnp.where(lane == e, cmb, 0.0), axis=-1, keepdims=True)
    y = y.astype(BF16).astype(F32) * w.astype(BF16).astype(F32)

    @pl.when(e == 0)
    def _():
        out_ref[sl, :] = y

    @pl.when(e > 0)
    def _():
        out_ref[sl, :] = out_ref[sl, :] + y


def _k6(xa_ref, routed_ref, h2_ref, wsg_ref, wsu_ref, wsd_ref, out_ref):
    h2 = h2_ref[...]
    gate = jax.lax.dot(h2, wsg_ref[...], preferred_element_type=F32)
    up = jax.lax.dot(h2, wsu_ref[...], preferred_element_type=F32)
    ymid = (gate * jax.nn.sigmoid(gate) * up).astype(BF16)
    y = jax.lax.dot(ymid, wsd_ref[...], preferred_element_type=F32)
    out_ref[...] = xa_ref[...] + routed_ref[...] + y


@jax.jit
def kernel(hidden_states, position_ids, ln1_w, Wq, Wk, Wv, Wo, ln2_w,
           W_router, expert_bias, Wg, Wu, Wd, Wsg, Wsu, Wsd):
    x = hidden_states.reshape(S, D)
    inv_freq = 1.0 / (THETA ** (jnp.arange(0, DH, 2, dtype=F32) / DH))
    ang = position_ids.reshape(S).astype(F32)[:, None] * inv_freq
    cos_t = jnp.concatenate([jnp.cos(ang), jnp.cos(ang)], axis=-1)
    sin_t = jnp.concatenate([jnp.sin(ang), jnp.sin(ang)], axis=-1)
    ln1 = ln1_w.reshape(1, D)
    ln2 = ln2_w.reshape(1, D)
    bias = expert_bias.reshape(1, E)
    wg, wu, wd = (w.astype(BF16) for w in (Wg, Wu, Wd))
    wsg, wsu, wsd = (w.astype(BF16) for w in (Wsg, Wsu, Wsd))

    q, k, v = pl.pallas_call(
        _k1,
        grid=(NTB,),
        in_specs=[
            pl.BlockSpec((BT, D), lambda t: (t, 0)),
            pl.BlockSpec((BT, DH), lambda t: (t, 0)),
            pl.BlockSpec((BT, DH), lambda t: (t, 0)),
            pl.BlockSpec((1, D), lambda t: (0, 0)),
            pl.BlockSpec((D, H * DH), lambda t: (0, 0)),
            pl.BlockSpec((D, HKV * DH), lambda t: (0, 0)),
            pl.BlockSpec((D, HKV * DH), lambda t: (0, 0)),
        ],
        out_specs=[
            pl.BlockSpec((BT, H * DH), lambda t: (t, 0)),
            pl.BlockSpec((BT, HKV * DH), lambda t: (t, 0)),
            pl.BlockSpec((BT, HKV * DH), lambda t: (t, 0)),
        ],
        out_shape=[
            jax.ShapeDtypeStruct((S, H * DH), BF16),
            jax.ShapeDtypeStruct((S, HKV * DH), BF16),
            jax.ShapeDtypeStruct((S, HKV * DH), BF16),
        ],
        compiler_params=pltpu.CompilerParams(
            vmem_limit_bytes=100 * 1024 * 1024),
    )(x, cos_t, sin_t, ln1, Wq, Wk, Wv)

    ctx = pl.pallas_call(
        _k2,
        grid=(H, S // BQ),
        in_specs=[
            pl.BlockSpec((BQ, DH), lambda h, t: (t, h)),
            pl.BlockSpec((S, DH), lambda h, t: (0, h // (H // HKV))),
            pl.BlockSpec((S, DH), lambda h, t: (0, h // (H // HKV))),
        ],
        out_specs=pl.BlockSpec((BQ, DH), lambda h, t: (t, h)),
        out_shape=jax.ShapeDtypeStruct((S, H * DH), BF16),
        compiler_params=pltpu.CompilerParams(
            vmem_limit_bytes=100 * 1024 * 1024),
    )(q, k, v)

    xa, h2, cmb = pl.pallas_call(
        _k3,
        grid=(NTB,),
        in_specs=[
            pl.BlockSpec((BT, H * DH), lambda t: (t, 0)),
            pl.BlockSpec((BT, D), lambda t: (t, 0)),
            pl.BlockSpec((H * DH, D), lambda t: (0, 0)),
            pl.BlockSpec((1, D), lambda t: (0, 0)),
            pl.BlockSpec((D, E), lambda t: (0, 0)),
            pl.BlockSpec((1, E), lambda t: (0, 0)),
        ],
        out_specs=[
            pl.BlockSpec((BT, D), lambda t: (t, 0)),
            pl.BlockSpec((BT, D), lambda t: (t, 0)),
            pl.BlockSpec((BT, E), lambda t: (t, 0)),
        ],
        out_shape=[
            jax.ShapeDtypeStruct((S, D), F32),
            jax.ShapeDtypeStruct((S, D), BF16),
            jax.ShapeDtypeStruct((S, E), F32),
        ],
        compiler_params=pltpu.CompilerParams(
            vmem_limit_bytes=100 * 1024 * 1024),
    )(ctx, x, Wo, ln2, W_router, bias)

    routed = pl.pallas_call(
        _k5,
        grid=(E, NTB),
        in_specs=[
            pl.BlockSpec((S, D), lambda e, t: (0, 0)),
            pl.BlockSpec((S, E), lambda e, t: (0, 0)),
            pl.BlockSpec((1, D, I), lambda e, t: (e, 0, 0)),
            pl.BlockSpec((1, D, I), lambda e, t: (e, 0, 0)),
            pl.BlockSpec((1, I, D), lambda e, t: (e, 0, 0)),
        ],
        out_specs=pl.BlockSpec((S, D), lambda e, t: (0, 0)),
        out_shape=jax.ShapeDtypeStruct((S, D), F32),
        compiler_params=pltpu.CompilerParams(
            dimension_semantics=("arbitrary", "arbitrary"),
            vmem_limit_bytes=100 * 1024 * 1024),
    )(h2, cmb, wg, wu, wd)

    out = pl.pallas_call(
        _k6,
        grid=(NTB,),
        in_specs=[
            pl.BlockSpec((BT, D), lambda t: (t, 0)),
            pl.BlockSpec((BT, D), lambda t: (t, 0)),
            pl.BlockSpec((BT, D), lambda t: (t, 0)),
            pl.BlockSpec((D, I), lambda t: (0, 0)),
            pl.BlockSpec((D, I), lambda t: (0, 0)),
            pl.BlockSpec((I, D), lambda t: (0, 0)),
        ],
        out_specs=pl.BlockSpec((BT, D), lambda t: (t, 0)),
        out_shape=jax.ShapeDtypeStruct((S, D), F32),
        compiler_params=pltpu.CompilerParams(
            vmem_limit_bytes=100 * 1024 * 1024),
    )(xa, routed, h2, wsg, wsu, wsd)

    return out.reshape(B, S, D)


# speedup vs baseline: 1.6561x; 1.1050x over previous
import functools
import math

import jax
import jax.numpy as jnp
import numpy as np
from jax.experimental import pallas as pl
from jax.experimental.pallas import tpu as pltpu

B, S, D = 1, 2048, 2048
H, HKV, DH = 16, 4, 128
E, K, I = 8, 2, 1024
EPS, THETA = 1e-05, 10000.0

BT = 256
NTB = S // BT

F32 = jnp.float32
BF16 = jnp.bfloat16


def _rms(xf, w):
    var = jnp.mean(xf * xf, axis=-1, keepdims=True)
    return w * (xf * jax.lax.rsqrt(var + EPS))


def _rope(x, cos_t, sin_t, width):
    n = width // DH
    cos = jnp.concatenate([cos_t] * n, axis=1)
    sin = jnp.concatenate([sin_t] * n, axis=1)
    lane = jax.lax.broadcasted_iota(jnp.int32, (BT, width), 1)
    left = jnp.roll(x, -64, axis=1)
    right = jnp.roll(x, 64, axis=1)
    rot = jnp.where((lane % 128) < 64, -left, right)
    return x * cos + rot * sin


def _k1(x_ref, cos_ref, sin_ref, ln1_ref, wq_ref, wk_ref, wv_ref,
        q_ref, k_ref, v_ref):
    h = _rms(x_ref[...].astype(F32), ln1_ref[...].astype(F32))
    cos_t = cos_ref[...]
    sin_t = sin_ref[...]
    q = jax.lax.dot(h, wq_ref[...], preferred_element_type=F32)
    k = jax.lax.dot(h, wk_ref[...], preferred_element_type=F32)
    v = jax.lax.dot(h, wv_ref[...], preferred_element_type=F32)
    q_ref[...] = _rope(q, cos_t, sin_t, H * DH).astype(BF16)
    k_ref[...] = _rope(k, cos_t, sin_t, HKV * DH).astype(BF16)
    v_ref[...] = v.astype(BF16)


BQ = 1024
BK = 1024
SCALE = np.float32(0.0883883461)


def _k2(q_ref, k_ref, v_ref, o_ref):
    for qb in range(S // BQ):
        qsl = pl.ds(qb * BQ, BQ)
        q = q_ref[qsl, :]
        qpos = qb * BQ + jax.lax.broadcasted_iota(jnp.int32, (BQ, BK), 0)
        m_prev = jnp.full((BQ, 1), -jnp.inf, F32)
        l_prev = jnp.zeros((BQ, 1), F32)
        acc = jnp.zeros((BQ, DH), F32)
        for kb in range(qb + 1):
            k = k_ref[kb * BK:(kb + 1) * BK, :]
            v = v_ref[kb * BK:(kb + 1) * BK, :]
            s = jax.lax.dot_general(q, k, (((1,), (1,)), ((), ())),
                                    preferred_element_type=F32)
            s = s * SCALE
            kpos = kb * BK + jax.lax.broadcasted_iota(jnp.int32, (BQ, BK), 1)
            s = jnp.where(kpos <= qpos, s, -1e9)
            m_cur = jnp.max(s, axis=-1, keepdims=True)
            m_new = jnp.maximum(m_prev, m_cur)
            delta = jnp.where(m_prev == m_new, 0.0, m_prev - m_new)
            p = jnp.exp(s - m_new)
            l_cur = jnp.sum(p, axis=-1, keepdims=True)
            corr_l = jnp.exp(delta) * l_prev
            l_new = corr_l + l_cur
            o = jax.lax.dot(p.astype(BF16), v, preferred_element_type=F32)
            o = o + acc * corr_l
            acc = o * (1.0 / l_new)
            m_prev, l_prev = m_new, l_new
        for _ in range(S // BK - (qb + 1)):
            acc = (acc * l_prev) * (1.0 / l_prev)
        o_ref[qsl, :] = acc.astype(BF16)


def _k3(ctx_ref, x_ref, wo_ref, ln2_ref, wr_ref, bias_ref,
        xa_ref, h2_ref, cmb_ref):
    attn = jax.lax.dot(ctx_ref[...], wo_ref[...], preferred_element_type=F32)
    xa = x_ref[...].astype(F32) + attn
    xa_ref[...] = xa
    h2 = _rms(xa, ln2_ref[...].astype(F32))
    h2_ref[...] = h2.astype(BF16)
    logits = jax.lax.dot(h2.astype(BF16), wr_ref[...].astype(BF16),
                         preferred_element_type=F32)
    sc = jax.nn.sigmoid(logits)
    biased = sc + bias_ref[...].astype(F32)
    lane = jax.lax.broadcasted_iota(jnp.int32, biased.shape, 1)

    def top1(b):
        m = jnp.max(b, axis=-1, keepdims=True)
        eq = b == m
        first = jnp.min(jnp.where(eq, lane, E), axis=-1, keepdims=True)
        return lane == first

    is0 = top1(biased)
    w0 = jnp.sum(jnp.where(is0, sc, 0.0), axis=-1, keepdims=True)
    is1 = top1(jnp.where(is0, -jnp.inf, biased))
    w1 = jnp.sum(jnp.where(is1, sc, 0.0), axis=-1, keepdims=True)
    wsum = w0 + w1 + 1e-9
    cmb_ref[...] = (jnp.where(is0, w0, 0.0) + jnp.where(is1, w1, 0.0)) / wsum


def _k5(h2_ref, cmb_ref, wg_ref, wu_ref, wd_ref, out_ref):
    e = pl.program_id(0)
    tb = pl.program_id(1)
    sl = pl.ds(tb * BT, BT)
    h2 = h2_ref[sl, :]
    gate = jax.lax.dot(h2, wg_ref[0], preferred_element_type=F32)
    up = jax.lax.dot(h2, wu_ref[0], preferred_element_type=F32)
    ymid = (gate * jax.nn.sigmoid(gate) * up).astype(BF16)
    y = jax.lax.dot(ymid, wd_ref[0], preferred_element_type=F32)
    cmb = cmb_ref[sl, :]
    lane = jax.lax.broadcasted_iota(jnp.int32, cmb.shape, 1)
    w = jnp.sum(jnp.where(lane == e, cmb, 0.0), axis=-1, keepdims=True)
    y = y.astype(BF16).astype(F32) * w.astype(BF16).astype(F32)

    @pl.when(e == 0)
    def _():
        out_ref[sl, :] = y

    @pl.when(e > 0)
    def _():
        out_ref[sl, :] = out_ref[sl, :] + y


def _k6(xa_ref, routed_ref, h2_ref, wsg_ref, wsu_ref, wsd_ref, out_ref):
    h2 = h2_ref[...]
    gate = jax.lax.dot(h2, wsg_ref[...], preferred_element_type=F32)
    up = jax.lax.dot(h2, wsu_ref[...], preferred_element_type=F32)
    ymid = (gate * jax.nn.sigmoid(gate) * up).astype(BF16)
    y = jax.lax.dot(ymid, wsd_ref[...], preferred_element_type=F32)
    out_ref[...] = xa_ref[...] + routed_ref[...] + y


@jax.jit
def kernel(hidden_states, position_ids, ln1_w, Wq, Wk, Wv, Wo, ln2_w,
           W_router, expert_bias, Wg, Wu, Wd, Wsg, Wsu, Wsd):
    x = hidden_states.reshape(S, D)
    inv_freq = 1.0 / (THETA ** (jnp.arange(0, DH, 2, dtype=F32) / DH))
    ang = position_ids.reshape(S).astype(F32)[:, None] * inv_freq
    cos_t = jnp.concatenate([jnp.cos(ang), jnp.cos(ang)], axis=-1)
    sin_t = jnp.concatenate([jnp.sin(ang), jnp.sin(ang)], axis=-1)
    ln1 = ln1_w.reshape(1, D)
    ln2 = ln2_w.reshape(1, D)
    bias = expert_bias.reshape(1, E)
    wg, wu, wd = (w.astype(BF16) for w in (Wg, Wu, Wd))
    wsg, wsu, wsd = (w.astype(BF16) for w in (Wsg, Wsu, Wsd))

    q, k, v = pl.pallas_call(
        _k1,
        grid=(NTB,),
        in_specs=[
            pl.BlockSpec((BT, D), lambda t: (t, 0)),
            pl.BlockSpec((BT, DH), lambda t: (t, 0)),
            pl.BlockSpec((BT, DH), lambda t: (t, 0)),
            pl.BlockSpec((1, D), lambda t: (0, 0)),
            pl.BlockSpec((D, H * DH), lambda t: (0, 0)),
            pl.BlockSpec((D, HKV * DH), lambda t: (0, 0)),
            pl.BlockSpec((D, HKV * DH), lambda t: (0, 0)),
        ],
        out_specs=[
            pl.BlockSpec((BT, H * DH), lambda t: (t, 0)),
            pl.BlockSpec((BT, HKV * DH), lambda t: (t, 0)),
            pl.BlockSpec((BT, HKV * DH), lambda t: (t, 0)),
        ],
        out_shape=[
            jax.ShapeDtypeStruct((S, H * DH), BF16),
            jax.ShapeDtypeStruct((S, HKV * DH), BF16),
            jax.ShapeDtypeStruct((S, HKV * DH), BF16),
        ],
        compiler_params=pltpu.CompilerParams(
            vmem_limit_bytes=100 * 1024 * 1024),
    )(x, cos_t, sin_t, ln1, Wq, Wk, Wv)

    ctx = pl.pallas_call(
        _k2,
        grid=(H,),
        in_specs=[
            pl.BlockSpec((S, DH), lambda h: (0, h)),
            pl.BlockSpec((S, DH), lambda h: (0, h // (H // HKV))),
            pl.BlockSpec((S, DH), lambda h: (0, h // (H // HKV))),
        ],
        out_specs=pl.BlockSpec((S, DH), lambda h: (0, h)),
        out_shape=jax.ShapeDtypeStruct((S, H * DH), BF16),
        compiler_params=pltpu.CompilerParams(
            vmem_limit_bytes=100 * 1024 * 1024),
    )(q, k, v)

    xa, h2, cmb = pl.pallas_call(
        _k3,
        grid=(NTB,),
        in_specs=[
            pl.BlockSpec((BT, H * DH), lambda t: (t, 0)),
            pl.BlockSpec((BT, D), lambda t: (t, 0)),
            pl.BlockSpec((H * DH, D), lambda t: (0, 0)),
            pl.BlockSpec((1, D), lambda t: (0, 0)),
            pl.BlockSpec((D, E), lambda t: (0, 0)),
            pl.BlockSpec((1, E), lambda t: (0, 0)),
        ],
        out_specs=[
            pl.BlockSpec((BT, D), lambda t: (t, 0)),
            pl.BlockSpec((BT, D), lambda t: (t, 0)),
            pl.BlockSpec((BT, E), lambda t: (t, 0)),
        ],
        out_shape=[
            jax.ShapeDtypeStruct((S, D), F32),
            jax.ShapeDtypeStruct((S, D), BF16),
            jax.ShapeDtypeStruct((S, E), F32),
        ],
        compiler_params=pltpu.CompilerParams(
            vmem_limit_bytes=100 * 1024 * 1024),
    )(ctx, x, Wo, ln2, W_router, bias)

    routed = pl.pallas_call(
        _k5,
        grid=(E, NTB),
        in_specs=[
            pl.BlockSpec((S, D), lambda e, t: (0, 0)),
            pl.BlockSpec((S, E), lambda e, t: (0, 0)),
            pl.BlockSpec((1, D, I), lambda e, t: (e, 0, 0)),
            pl.BlockSpec((1, D, I), lambda e, t: (e, 0, 0)),
            pl.BlockSpec((1, I, D), lambda e, t: (e, 0, 0)),
        ],
        out_specs=pl.BlockSpec((S, D), lambda e, t: (0, 0)),
        out_shape=jax.ShapeDtypeStruct((S, D), F32),
        compiler_params=pltpu.CompilerParams(
            dimension_semantics=("arbitrary", "arbitrary"),
            vmem_limit_bytes=100 * 1024 * 1024),
    )(h2, cmb, wg, wu, wd)

    out = pl.pallas_call(
        _k6,
        grid=(NTB,),
        in_specs=[
            pl.BlockSpec((BT, D), lambda t: (t, 0)),
            pl.BlockSpec((BT, D), lambda t: (t, 0)),
            pl.BlockSpec((BT, D), lambda t: (t, 0)),
            pl.BlockSpec((D, I), lambda t: (0, 0)),
            pl.BlockSpec((D, I), lambda t: (0, 0)),
            pl.BlockSpec((I, D), lambda t: (0, 0)),
        ],
        out_specs=pl.BlockSpec((BT, D), lambda t: (t, 0)),
        out_shape=jax.ShapeDtypeStruct((S, D), F32),
        compiler_params=pltpu.CompilerParams(
            vmem_limit_bytes=100 * 1024 * 1024),
    )(xa, routed, h2, wsg, wsu, wsd)

    return out.reshape(B, S, D)


# speedup vs baseline: 1.6567x; 1.0004x over previous
import jax
import jax.numpy as jnp
import numpy as np
from jax.experimental import pallas as pl
from jax.experimental.pallas import tpu as pltpu

B, S, D = 1, 2048, 2048
H, HKV, DH = 16, 4, 128
E, K, I = 8, 2, 1024
EPS, THETA = 1e-05, 10000.0

BT = 256
NTB = S // BT

F32 = jnp.float32
BF16 = jnp.bfloat16


def _rms(xf, w):
    var = jnp.mean(xf * xf, axis=-1, keepdims=True)
    return w * (xf * jax.lax.rsqrt(var + EPS))


def _rope(x, cos_t, sin_t, width):
    n = width // DH
    cos = jnp.concatenate([cos_t] * n, axis=1)
    sin = jnp.concatenate([sin_t] * n, axis=1)
    lane = jax.lax.broadcasted_iota(jnp.int32, (BT, width), 1)
    left = jnp.roll(x, -64, axis=1)
    right = jnp.roll(x, 64, axis=1)
    rot = jnp.where((lane % 128) < 64, -left, right)
    return x * cos + rot * sin


def _k1(x_ref, cos_ref, sin_ref, ln1_ref, wq_ref, wk_ref, wv_ref,
        q_ref, k_ref, v_ref):
    h = _rms(x_ref[...].astype(F32), ln1_ref[...].astype(F32))
    cos_t = cos_ref[...]
    sin_t = sin_ref[...]
    q = jax.lax.dot(h, wq_ref[...], preferred_element_type=F32)
    k = jax.lax.dot(h, wk_ref[...], preferred_element_type=F32)
    v = jax.lax.dot(h, wv_ref[...], preferred_element_type=F32)
    q_ref[...] = _rope(q, cos_t, sin_t, H * DH).astype(BF16)
    k_ref[...] = _rope(k, cos_t, sin_t, HKV * DH).astype(BF16)
    v_ref[...] = v.astype(BF16)


BQ = 1024
BK = 1024
SCALE = np.float32(0.0883883461)


def _k2(q_ref, k_ref, v_ref, o_ref):
    for qb in range(S // BQ):
        qsl = pl.ds(qb * BQ, BQ)
        q = q_ref[qsl, :]
        qpos = qb * BQ + jax.lax.broadcasted_iota(jnp.int32, (BQ, BK), 0)
        m_prev = jnp.full((BQ, 1), -jnp.inf, F32)
        l_prev = jnp.zeros((BQ, 1), F32)
        acc = jnp.zeros((BQ, DH), F32)
        for kb in range(qb + 1):
            k = k_ref[kb * BK:(kb + 1) * BK, :]
            v = v_ref[kb * BK:(kb + 1) * BK, :]
            s = jax.lax.dot_general(q, k, (((1,), (1,)), ((), ())),
                                    preferred_element_type=F32)
            s = s * SCALE
            kpos = kb * BK + jax.lax.broadcasted_iota(jnp.int32, (BQ, BK), 1)
            s = jnp.where(kpos <= qpos, s, -1e9)
            m_cur = jnp.max(s, axis=-1, keepdims=True)
            m_new = jnp.maximum(m_prev, m_cur)
            delta = jnp.where(m_prev == m_new, 0.0, m_prev - m_new)
            p = jnp.exp(s - m_new)
            l_cur = jnp.sum(p, axis=-1, keepdims=True)
            corr_l = jnp.exp(delta) * l_prev
            l_new = corr_l + l_cur
            o = jax.lax.dot(p.astype(BF16), v, preferred_element_type=F32)
            o = o + acc * corr_l
            acc = o * (1.0 / l_new)
            m_prev, l_prev = m_new, l_new
        for _ in range(S // BK - (qb + 1)):
            acc = (acc * l_prev) * (1.0 / l_prev)
        o_ref[qsl, :] = acc.astype(BF16)


def _k3(ctx_ref, x_ref, wo_ref, ln2_ref, wr_ref, bias_ref,
        xa_ref, h2_ref, cmb_ref):
    attn = jax.lax.dot(ctx_ref[...], wo_ref[...], preferred_element_type=F32)
    xa = x_ref[...].astype(F32) + attn
    xa_ref[...] = xa
    h2 = _rms(xa, ln2_ref[...].astype(F32))
    h2_ref[...] = h2.astype(BF16)
    logits = jax.lax.dot(h2.astype(BF16), wr_ref[...].astype(BF16),
                         preferred_element_type=F32)
    sc = jax.nn.sigmoid(logits)
    biased = sc + bias_ref[...].astype(F32)
    lane = jax.lax.broadcasted_iota(jnp.int32, biased.shape, 1)

    def top1(b):
        m = jnp.max(b, axis=-1, keepdims=True)
        eq = b == m
        first = jnp.min(jnp.where(eq, lane, E), axis=-1, keepdims=True)
        return lane == first

    is0 = top1(biased)
    w0 = jnp.sum(jnp.where(is0, sc, 0.0), axis=-1, keepdims=True)
    is1 = top1(jnp.where(is0, -jnp.inf, biased))
    w1 = jnp.sum(jnp.where(is1, sc, 0.0), axis=-1, keepdims=True)
    wsum = w0 + w1 + 1e-9
    cmb_ref[...] = (jnp.where(is0, w0, 0.0) + jnp.where(is1, w1, 0.0)) / wsum


def _k5(h2_ref, cmb_ref, wg_ref, wu_ref, wd_ref, out_ref):
    e = pl.program_id(0)
    tb = pl.program_id(1)
    sl = pl.ds(tb * BT, BT)
    h2 = h2_ref[sl, :]
    gate = jax.lax.dot(h2, wg_ref[0], preferred_element_type=F32)
    up = jax.lax.dot(h2, wu_ref[0], preferred_element_type=F32)
    ymid = (gate * jax.nn.sigmoid(gate) * up).astype(BF16)
    y = jax.lax.dot(ymid, wd_ref[0], preferred_element_type=F32)
    cmb = cmb_ref[sl, :]
    lane = jax.lax.broadcasted_iota(jnp.int32, cmb.shape, 1)
    w = jnp.sum(jnp.where(lane == e, cmb, 0.0), axis=-1, keepdims=True)
    y = y.astype(BF16).astype(F32) * w.astype(BF16).astype(F32)

    @pl.when(e == 0)
    def _():
        out_ref[sl, :] = y

    @pl.when(e > 0)
    def _():
        out_ref[sl, :] = out_ref[sl, :] + y


def _k6(xa_ref, routed_ref, h2_ref, wsg_ref, wsu_ref, wsd_ref, out_ref):
    h2 = h2_ref[...]
    gate = jax.lax.dot(h2, wsg_ref[...], preferred_element_type=F32)
    up = jax.lax.dot(h2, wsu_ref[...], preferred_element_type=F32)
    ymid = (gate * jax.nn.sigmoid(gate) * up).astype(BF16)
    y = jax.lax.dot(ymid, wsd_ref[...], preferred_element_type=F32)
    out_ref[...] = xa_ref[...] + routed_ref[...] + y


@jax.jit
def kernel(hidden_states, position_ids, ln1_w, Wq, Wk, Wv, Wo, ln2_w,
           W_router, expert_bias, Wg, Wu, Wd, Wsg, Wsu, Wsd):
    x = hidden_states.reshape(S, D)
    inv_freq = 1.0 / (THETA ** (jnp.arange(0, DH, 2, dtype=F32) / DH))
    ang = position_ids.reshape(S).astype(F32)[:, None] * inv_freq
    cos_t = jnp.concatenate([jnp.cos(ang), jnp.cos(ang)], axis=-1)
    sin_t = jnp.concatenate([jnp.sin(ang), jnp.sin(ang)], axis=-1)
    ln1 = ln1_w.reshape(1, D)
    ln2 = ln2_w.reshape(1, D)
    bias = expert_bias.reshape(1, E)
    wg, wu, wd = (w.astype(BF16) for w in (Wg, Wu, Wd))
    wsg, wsu, wsd = (w.astype(BF16) for w in (Wsg, Wsu, Wsd))

    q, k, v = pl.pallas_call(
        _k1,
        grid=(NTB,),
        in_specs=[
            pl.BlockSpec((BT, D), lambda t: (t, 0)),
            pl.BlockSpec((BT, DH), lambda t: (t, 0)),
            pl.BlockSpec((BT, DH), lambda t: (t, 0)),
            pl.BlockSpec((1, D), lambda t: (0, 0)),
            pl.BlockSpec((D, H * DH), lambda t: (0, 0)),
            pl.BlockSpec((D, HKV * DH), lambda t: (0, 0)),
            pl.BlockSpec((D, HKV * DH), lambda t: (0, 0)),
        ],
        out_specs=[
            pl.BlockSpec((BT, H * DH), lambda t: (t, 0)),
            pl.BlockSpec((BT, HKV * DH), lambda t: (t, 0)),
            pl.BlockSpec((BT, HKV * DH), lambda t: (t, 0)),
        ],
        out_shape=[
            jax.ShapeDtypeStruct((S, H * DH), BF16),
            jax.ShapeDtypeStruct((S, HKV * DH), BF16),
            jax.ShapeDtypeStruct((S, HKV * DH), BF16),
        ],
        compiler_params=pltpu.CompilerParams(
            vmem_limit_bytes=100 * 1024 * 1024),
    )(x, cos_t, sin_t, ln1, Wq, Wk, Wv)

    ctx = pl.pallas_call(
        _k2,
        grid=(H,),
        in_specs=[
            pl.BlockSpec((S, DH), lambda h: (0, h)),
            pl.BlockSpec((S, DH), lambda h: (0, h // (H // HKV))),
            pl.BlockSpec((S, DH), lambda h: (0, h // (H // HKV))),
        ],
        out_specs=pl.BlockSpec((S, DH), lambda h: (0, h)),
        out_shape=jax.ShapeDtypeStruct((S, H * DH), BF16),
        compiler_params=pltpu.CompilerParams(
            vmem_limit_bytes=100 * 1024 * 1024),
    )(q, k, v)

    xa, h2, cmb = pl.pallas_call(
        _k3,
        grid=(NTB,),
        in_specs=[
            pl.BlockSpec((BT, H * DH), lambda t: (t, 0)),
            pl.BlockSpec((BT, D), lambda t: (t, 0)),
            pl.BlockSpec((H * DH, D), lambda t: (0, 0)),
            pl.BlockSpec((1, D), lambda t: (0, 0)),
            pl.BlockSpec((D, E), lambda t: (0, 0)),
            pl.BlockSpec((1, E), lambda t: (0, 0)),
        ],
        out_specs=[
            pl.BlockSpec((BT, D), lambda t: (t, 0)),
            pl.BlockSpec((BT, D), lambda t: (t, 0)),
            pl.BlockSpec((BT, E), lambda t: (t, 0)),
        ],
        out_shape=[
            jax.ShapeDtypeStruct((S, D), F32),
            jax.ShapeDtypeStruct((S, D), BF16),
            jax.ShapeDtypeStruct((S, E), F32),
        ],
        compiler_params=pltpu.CompilerParams(
            vmem_limit_bytes=100 * 1024 * 1024),
    )(ctx, x, Wo, ln2, W_router, bias)

    routed = pl.pallas_call(
        _k5,
        grid=(E, NTB),
        in_specs=[
            pl.BlockSpec((S, D), lambda e, t: (0, 0)),
            pl.BlockSpec((S, E), lambda e, t: (0, 0)),
            pl.BlockSpec((1, D, I), lambda e, t: (e, 0, 0)),
            pl.BlockSpec((1, D, I), lambda e, t: (e, 0, 0)),
            pl.BlockSpec((1, I, D), lambda e, t: (e, 0, 0)),
        ],
        out_specs=pl.BlockSpec((S, D), lambda e, t: (0, 0)),
        out_shape=jax.ShapeDtypeStruct((S, D), F32),
        compiler_params=pltpu.CompilerParams(
            dimension_semantics=("arbitrary", "arbitrary"),
            vmem_limit_bytes=100 * 1024 * 1024),
    )(h2, cmb, wg, wu, wd)

    out = pl.pallas_call(
        _k6,
        grid=(NTB,),
        in_specs=[
            pl.BlockSpec((BT, D), lambda t: (t, 0)),
            pl.BlockSpec((BT, D), lambda t: (t, 0)),
            pl.BlockSpec((BT, D), lambda t: (t, 0)),
            pl.BlockSpec((D, I), lambda t: (0, 0)),
            pl.BlockSpec((D, I), lambda t: (0, 0)),
            pl.BlockSpec((I, D), lambda t: (0, 0)),
        ],
        out_specs=pl.BlockSpec((BT, D), lambda t: (t, 0)),
        out_shape=jax.ShapeDtypeStruct((S, D), F32),
        compiler_params=pltpu.CompilerParams(
            vmem_limit_bytes=100 * 1024 * 1024),
    )(xa, routed, h2, wsg, wsu, wsd)

    return out.reshape(B, S, D)
